# baseline (speedup 1.0000x reference)
import functools
import jax
import jax.numpy as jnp
from jax import lax
from jax.experimental import pallas as pl
from jax.experimental.pallas import tpu as pltpu
from jax.experimental.pallas import tpu_sc as plsc

N_NODES = 10000
N_EDGES = 160000
DIM = 256
HIDDEN = 512
QUART = DIM // 4
PADC = 16

N_SUB = 16
NP = 10240
CHUNK = 128
EPAD = 1280 * CHUNK
NCH = EPAD // CHUNK // N_SUB
RPT = NP // N_SUB
RCH = RPT // 5
PAD_DST = NP - 8

_MESH = plsc.VectorSubcoreMesh(core_axis_name="c", subcore_axis_name="s")
_SC_PARAMS = pltpu.CompilerParams(use_tc_tiling_on_sc=False)


def _zero_fill_2d(ref, nrows, ncols):
    z = jnp.zeros((16,), jnp.float32)

    def row(r, carry):
        for j in range(ncols // 16):
            ref[r, pl.ds(j * 16, 16)] = z
        return carry

    lax.fori_loop(0, nrows, row, 0)


def _ring(nch, fill, gstart, gwait, sstart, swait):
    for cc in range(4):
        fill(cc, cc)
        gstart(cc, cc)
        if cc >= 2:
            gwait(cc - 2)
            sstart(cc - 2, cc - 2)

    def body(g, carry):
        for b in range(4):
            c = 4 * g + 4 + b
            swait(b)
            fill(c, b)
            gstart(c, b)
            b2 = (b + 2) % 4
            gwait(b2)
            sstart(c - 2, b2)
        return carry

    lax.fori_loop(0, (nch - 4) // 4, body, 0)
    gwait(2)
    sstart(nch - 2, 2)
    gwait(3)
    sstart(nch - 1, 3)
    for b in range(4):
        swait(b)


@functools.partial(
    pl.kernel,
    out_type=[
        jax.ShapeDtypeStruct((4 * NP, QUART), jnp.float32),
        jax.ShapeDtypeStruct((NP, PADC), jnp.float32),
    ],
    mesh=_MESH,
    compiler_params=_SC_PARAMS,
    scratch_types=[
        pltpu.VMEM((NCH, CHUNK), jnp.int32),
        pltpu.VMEM((NCH, CHUNK), jnp.int32),
        pltpu.VMEM((CHUNK,), jnp.int32),
        pltpu.VMEM((CHUNK,), jnp.int32),
        pltpu.VMEM((CHUNK,), jnp.int32),
        pltpu.VMEM((CHUNK,), jnp.int32),
        pltpu.VMEM((CHUNK, QUART), jnp.float32),
        pltpu.VMEM((CHUNK, QUART), jnp.float32),
        pltpu.VMEM((CHUNK, QUART), jnp.float32),
        pltpu.VMEM((CHUNK, QUART), jnp.float32),
        pltpu.VMEM((CHUNK, PADC), jnp.float32),
        pltpu.VMEM((RCH, QUART), jnp.float32),
        pltpu.VMEM((RPT, PADC), jnp.float32),
        pltpu.VMEM_SHARED((NP, QUART), jnp.float32),
        pltpu.VMEM_SHARED((NP, PADC), jnp.float32),
        pltpu.SemaphoreType.DMA,
        pltpu.SemaphoreType.DMA,
        pltpu.SemaphoreType.DMA,
        pltpu.SemaphoreType.DMA,
        pltpu.SemaphoreType.DMA,
        pltpu.SemaphoreType.DMA,
        pltpu.SemaphoreType.DMA,
        pltpu.SemaphoreType.DMA,
    ],
)
def _agg1(xs_hbm, src2_hbm, dst2_hbm, out_hbm, cnt_hbm,
          srcb_v, dstb_v, i0, i1, i2, i3, r0, r1, r2, r3, ones_v,
          tmp_v, tmp16_v, acc_s, cnt_s,
          g0, g1, g2, g3, s0, s1, s2, s3):
    c = lax.axis_index("c")
    s = lax.axis_index("s")
    idx = [i0, i1, i2, i3]
    rows = [r0, r1, r2, r3]
    gsem = [g0, g1, g2, g3]
    ssem = [s0, s1, s2, s3]

    _zero_fill_2d(tmp_v, RCH, QUART)
    _zero_fill_2d(tmp16_v, RPT, PADC)
    one = jnp.ones((16,), jnp.float32)

    def ones_row(r, carry):
        ones_v[r, pl.ds(0, PADC)] = one
        return carry

    lax.fori_loop(0, CHUNK, ones_row, 0)

    pltpu.sync_copy(src2_hbm.at[pl.ds(s * NCH, NCH)], srcb_v)
    pltpu.sync_copy(dst2_hbm.at[pl.ds(s * NCH, NCH)], dstb_v)

    for p in range(2):
        qq = 2 * p + c

        def fill(ci, b):
            for j in range(CHUNK // 16):
                idx[b][pl.ds(j * 16, 16)] = (
                    srcb_v[ci, pl.ds(j * 16, 16)] * 4 + qq)

        def gstart(ci, b):
            pltpu.async_copy(xs_hbm.at[idx[b]], rows[b], gsem[b])

        def gwait(b):
            pltpu.make_async_copy(xs_hbm.at[idx[b]], rows[b], gsem[b]).wait()

        def sstart(ci, b):
            pltpu.async_copy(rows[b], acc_s.at[dstb_v.at[ci]], ssem[b],
                             add=True)
            if p == 0:
                pltpu.async_copy(ones_v, cnt_s.at[dstb_v.at[ci]], ssem[b],
                                 add=True)

        def swait(b):
            pltpu.make_async_copy(rows[b], acc_s.at[dstb_v.at[0]],
                                  ssem[b]).wait()
            if p == 0:
                pltpu.make_async_copy(ones_v, cnt_s.at[dstb_v.at[0]],
                                      ssem[b]).wait()

        for k in range(5):
            pltpu.sync_copy(tmp_v, acc_s.at[pl.ds(s * RPT + k * RCH, RCH)])
        if p == 0:
            pltpu.sync_copy(tmp16_v, cnt_s.at[pl.ds(s * RPT, RPT)])
        plsc.subcore_barrier()

        _ring(NCH, fill, gstart, gwait, sstart, swait)
        plsc.subcore_barrier()

        for k in range(5):
            r0_ = s * RPT + k * RCH
            pltpu.sync_copy(acc_s.at[pl.ds(r0_, RCH)], tmp_v)
            pltpu.sync_copy(tmp_v, out_hbm.at[pl.ds(qq * NP + r0_, RCH)])
        if p == 0:
            _zero_fill_2d(tmp_v, RCH, QUART)

            @pl.when(c == 0)
            def _():
                pltpu.sync_copy(cnt_s.at[pl.ds(s * RPT, RPT)], tmp16_v)
                pltpu.sync_copy(tmp16_v, cnt_hbm.at[pl.ds(s * RPT, RPT)])


_BM = 640


def _dense_body(x_ref, s0_ref, s1_ref, s2_ref, s3_ref, cnt_ref,
                w1l_ref, w1r_ref, b1_ref, w2l_ref, w2r_ref, b2_ref,
                p_ref, q_ref):
    inv = 1.0 / jnp.maximum(cnt_ref[:, :1], 1.0)
    mean = jnp.concatenate(
        [s0_ref[...], s1_ref[...], s2_ref[...], s3_ref[...]], axis=1) * inv
    h = (jnp.dot(mean, w1l_ref[...], preferred_element_type=jnp.float32)
         + jnp.dot(x_ref[...], w1r_ref[...], preferred_element_type=jnp.float32)
         + b1_ref[...])
    h = jnp.maximum(h, 0.0)
    p_ref[...] = jnp.dot(h, w2l_ref[...], preferred_element_type=jnp.float32)
    q_ref[...] = (jnp.dot(h, w2r_ref[...], preferred_element_type=jnp.float32)
                  + b2_ref[...])


def _make_sum_spec(q):
    return pl.BlockSpec((_BM, QUART), lambda i, q=q: (q * (NP // _BM) + i, 0))


_dense = pl.pallas_call(
    _dense_body,
    grid=(NP // _BM,),
    in_specs=[
        pl.BlockSpec((_BM, DIM), lambda i: (i, 0)),
        _make_sum_spec(0),
        _make_sum_spec(1),
        _make_sum_spec(2),
        _make_sum_spec(3),
        pl.BlockSpec((_BM, PADC), lambda i: (i, 0)),
        pl.BlockSpec((DIM, HIDDEN), lambda i: (0, 0)),
        pl.BlockSpec((DIM, HIDDEN), lambda i: (0, 0)),
        pl.BlockSpec((1, HIDDEN), lambda i: (0, 0)),
        pl.BlockSpec((HIDDEN, PADC), lambda i: (0, 0)),
        pl.BlockSpec((HIDDEN, PADC), lambda i: (0, 0)),
        pl.BlockSpec((1, PADC), lambda i: (0, 0)),
    ],
    out_specs=[
        pl.BlockSpec((_BM, PADC), lambda i: (i, 0)),
        pl.BlockSpec((_BM, PADC), lambda i: (i, 0)),
    ],
    out_shape=[
        jax.ShapeDtypeStruct((NP, PADC), jnp.float32),
        jax.ShapeDtypeStruct((NP, PADC), jnp.float32),
    ],
)


@functools.partial(
    pl.kernel,
    out_type=jax.ShapeDtypeStruct((NP, PADC), jnp.float32),
    mesh=_MESH,
    compiler_params=_SC_PARAMS,
    scratch_types=[
        pltpu.VMEM((NCH, CHUNK), jnp.int32),
        pltpu.VMEM((NCH, CHUNK), jnp.int32),
        pltpu.VMEM((CHUNK, PADC), jnp.float32),
        pltpu.VMEM((CHUNK, PADC), jnp.float32),
        pltpu.VMEM((CHUNK, PADC), jnp.float32),
        pltpu.VMEM((CHUNK, PADC), jnp.float32),
        pltpu.VMEM((RPT, PADC), jnp.float32),
        pltpu.VMEM((RPT, PADC), jnp.float32),
        pltpu.VMEM((RPT, PADC), jnp.float32),
        pltpu.VMEM_SHARED((NP, PADC), jnp.float32),
        pltpu.SemaphoreType.DMA,
        pltpu.SemaphoreType.DMA,
        pltpu.SemaphoreType.DMA,
        pltpu.SemaphoreType.DMA,
        pltpu.SemaphoreType.DMA,
        pltpu.SemaphoreType.DMA,
        pltpu.SemaphoreType.DMA,
        pltpu.SemaphoreType.DMA,
    ],
)
def _agg2(p_hbm, q_hbm, cnt_hbm, src2_hbm, dst2_hbm, out_hbm,
          srcb_v, dstb_v, r0, r1, r2, r3, a_v, c_v, q_v, acc_s,
          g0, g1, g2, g3, s0, s1, s2, s3):
    c = lax.axis_index("c")
    s = lax.axis_index("s")
    rows = [r0, r1, r2, r3]
    gsem = [g0, g1, g2, g3]
    ssem = [s0, s1, s2, s3]

    @pl.when(c == 0)
    def _():
        _zero_fill_2d(a_v, RPT, PADC)
        pltpu.sync_copy(a_v, acc_s.at[pl.ds(s * RPT, RPT)])
        pltpu.sync_copy(src2_hbm.at[pl.ds(s * NCH, NCH)], srcb_v)
        pltpu.sync_copy(dst2_hbm.at[pl.ds(s * NCH, NCH)], dstb_v)
        plsc.subcore_barrier()

        def fill(ci, b):
            pass

        def gstart(ci, b):
            pltpu.async_copy(p_hbm.at[srcb_v.at[ci]], rows[b], gsem[b])

        def gwait(b):
            pltpu.make_async_copy(p_hbm.at[srcb_v.at[0]], rows[b],
                                  gsem[b]).wait()

        def sstart(ci, b):
            pltpu.async_copy(rows[b], acc_s.at[dstb_v.at[ci]], ssem[b],
                             add=True)

        def swait(b):
            pltpu.make_async_copy(rows[b], acc_s.at[dstb_v.at[0]],
                                  ssem[b]).wait()

        _ring(NCH, fill, gstart, gwait, sstart, swait)
        plsc.subcore_barrier()

        r0_ = s * RPT
        pltpu.sync_copy(acc_s.at[pl.ds(r0_, RPT)], a_v)
        pltpu.sync_copy(cnt_hbm.at[pl.ds(r0_, RPT)], c_v)
        pltpu.sync_copy(q_hbm.at[pl.ds(r0_, RPT)], q_v)

        def row(r, carry):
            agg = a_v[r, pl.ds(0, PADC)]
            cc = jnp.maximum(c_v[r, pl.ds(0, PADC)], 1.0)
            q_v[r, pl.ds(0, PADC)] = agg / cc + q_v[r, pl.ds(0, PADC)]
            return carry

        lax.fori_loop(0, RPT, row, 0)
        pltpu.sync_copy(q_v, out_hbm.at[pl.ds(r0_, RPT)])


def kernel(x, edge_index, W1l, b1l, W1r, W2l, b2l, W2r):
    src = edge_index[0].astype(jnp.int32)
    dst = edge_index[1].astype(jnp.int32)

    pad = EPAD - N_EDGES
    srcp = jnp.concatenate([src, jnp.zeros((pad,), jnp.int32)])
    dstp = jnp.concatenate([dst, jnp.full((pad,), PAD_DST, jnp.int32)])
    xs = x.reshape(4 * N_NODES, QUART)
    src2 = srcp.reshape(EPAD // CHUNK, CHUNK)
    dst2 = dstp.reshape(EPAD // CHUNK, CHUNK)

    summed4, cnt = _agg1(xs, src2, dst2)

    nc = W2l.shape[0]
    padw = jnp.zeros((PADC - nc, HIDDEN), jnp.float32)
    w2l_t = jnp.concatenate([W2l, padw], axis=0).T
    w2r_t = jnp.concatenate([W2r, padw], axis=0).T
    b2p = jnp.concatenate([b2l, jnp.zeros((PADC - nc,), jnp.float32)])[None]

    p16, q16 = _dense(x, summed4, summed4, summed4, summed4, cnt,
                      W1l.T, W1r.T, b1l[None], w2l_t, w2r_t, b2p)
    out16 = _agg2(p16, q16, cnt, src2, dst2)
    return out16[:N_NODES, :nc]
